# Initial kernel scaffold; baseline (speedup 1.0000x reference)
import jax
import jax.numpy as jnp
from jax.experimental import pallas as pl


def kernel(wrong_features, right_features, ref_boxes, curr_boxes, wq, bq, wk, bk, wv, bv, wo, bo, g1, b1, wgen_w, wgen_b, g2, b2, wf1, bf1, wf2, bf2, g3, b3):
    raise NotImplementedError("write your pallas kernel here")


# speedup vs baseline: 45.6269x; 45.6269x over previous
import functools

import jax
import jax.numpy as jnp
import numpy as np
from jax.experimental import pallas as pl

K, C, R, NH, FFN, GROUPS, KS, M = 1024, 128, 7, 8, 1024, 4, 3, 64
D = C * R * R
P = R * R
CG = C // GROUPS
NT = KS * KS
BN = 896
NB = D // BN
BM = 128
MB = K // BM
F32 = jnp.float32


def _mm_bias_kernel(x_ref, w_ref, b_ref, o_ref):
    acc = jnp.dot(x_ref[...], w_ref[...], preferred_element_type=F32)
    o_ref[...] = acc + b_ref[...]


def _mm_bias(x, w, b):
    n_blocks = w.shape[1] // BN
    return pl.pallas_call(
        _mm_bias_kernel,
        grid=(n_blocks, MB),
        in_specs=[
            pl.BlockSpec((BM, x.shape[1]), lambda n, m: (m, 0)),
            pl.BlockSpec((w.shape[0], BN), lambda n, m: (0, n)),
            pl.BlockSpec((1, BN), lambda n, m: (0, n)),
        ],
        out_specs=pl.BlockSpec((BM, BN), lambda n, m: (m, n)),
        out_shape=jax.ShapeDtypeStruct((x.shape[0], w.shape[1]), F32),
    )(x, w, b)


def _mm_bias_res_kernel(x_ref, w_ref, b_ref, r_ref, o_ref):
    acc = jnp.dot(x_ref[...], w_ref[...], preferred_element_type=F32)
    o_ref[...] = acc + b_ref[...] + r_ref[...]


def _mm_bias_res(x, w, b, res):
    n_blocks = w.shape[1] // BN
    return pl.pallas_call(
        _mm_bias_res_kernel,
        grid=(n_blocks, MB),
        in_specs=[
            pl.BlockSpec((BM, x.shape[1]), lambda n, m: (m, 0)),
            pl.BlockSpec((w.shape[0], BN), lambda n, m: (0, n)),
            pl.BlockSpec((1, BN), lambda n, m: (0, n)),
            pl.BlockSpec((BM, BN), lambda n, m: (m, n)),
        ],
        out_specs=pl.BlockSpec((BM, BN), lambda n, m: (m, n)),
        out_shape=jax.ShapeDtypeStruct((x.shape[0], w.shape[1]), F32),
    )(x, w, b, res)


def _ln_gap_kernel(x_ref, g_ref, b_ref, y_ref, gap_ref):
    x = x_ref[...]
    mu = jnp.mean(x, axis=(1, 2), keepdims=True)
    xc = x - mu
    var = jnp.mean(xc * xc, axis=(1, 2), keepdims=True)
    y = xc * jax.lax.rsqrt(var + 1e-5) * g_ref[...] + b_ref[...]
    y_ref[...] = y
    gap_ref[...] = jnp.mean(y, axis=2)


def _ln_gap(x3, g, b):
    return pl.pallas_call(
        _ln_gap_kernel,
        grid=(MB,),
        in_specs=[
            pl.BlockSpec((BM, C, P), lambda m: (m, 0, 0)),
            pl.BlockSpec((1, C, P), lambda m: (0, 0, 0)),
            pl.BlockSpec((1, C, P), lambda m: (0, 0, 0)),
        ],
        out_specs=[
            pl.BlockSpec((BM, C, P), lambda m: (m, 0, 0)),
            pl.BlockSpec((BM, C), lambda m: (m, 0)),
        ],
        out_shape=[
            jax.ShapeDtypeStruct((K, C, P), F32),
            jax.ShapeDtypeStruct((K, C), F32),
        ],
    )(x3, g, b)


DW_BN = 2304


def _gen_kernel(g_ref, w_ref, b_ref, o_ref):
    o_ref[...] = jnp.dot(g_ref[...], w_ref[...],
                         preferred_element_type=F32) + b_ref[...]


def _gen(gap, wgp, bgp):
    n_blocks = wgp.shape[1] // DW_BN
    return pl.pallas_call(
        _gen_kernel,
        grid=(n_blocks,),
        in_specs=[
            pl.BlockSpec((K, C), lambda n: (0, 0)),
            pl.BlockSpec((C, DW_BN), lambda n: (0, n)),
            pl.BlockSpec((1, DW_BN), lambda n: (0, n)),
        ],
        out_specs=pl.BlockSpec((K, DW_BN), lambda n: (0, n)),
        out_shape=jax.ShapeDtypeStruct((K, wgp.shape[1]), F32),
    )(gap, wgp, bgp)


CONV_S = 8


def _conv_ln_kernel(x_ref, dw_ref, t_ref, g_ref, b_ref, o_ref):
    S = CONV_S
    x = x_ref[...]
    xs2 = x.reshape(S * C, P)
    shs = [jnp.dot(xs2, t_ref[t], preferred_element_type=F32).reshape(S, C, P)
           for t in range(NT)]
    patches = jnp.stack(shs, axis=1).reshape(S, NT * C, P)
    dwv = dw_ref[...]
    wtile = jnp.broadcast_to(dwv[:, :, None, :, :],
                             (S, NT, GROUPS, CG, C)).reshape(S, NT, C, C)
    ci = jax.lax.broadcasted_iota(jnp.int32, (C, C), 0)
    co = jax.lax.broadcasted_iota(jnp.int32, (C, C), 1)
    mask = (ci // CG == co // CG).astype(F32)
    wd = (wtile * mask).reshape(S, NT * C, C)
    outs = []
    for s in range(S):
        outs.append(jax.lax.dot_general(
            wd[s], patches[s], (((0,), (0,)), ((), ())),
            preferred_element_type=F32))
    dc = jnp.stack(outs, axis=0)
    y = dc + x
    mu = jnp.mean(y, axis=(1, 2), keepdims=True)
    yc = y - mu
    var = jnp.mean(yc * yc, axis=(1, 2), keepdims=True)
    o_ref[...] = yc * jax.lax.rsqrt(var + 1e-5) * g_ref[...] + b_ref[...]


def _conv_ln(x3, dw4, tmats, g2v, b2v):
    return pl.pallas_call(
        _conv_ln_kernel,
        grid=(K // CONV_S,),
        in_specs=[
            pl.BlockSpec((CONV_S, C, P), lambda m: (m, 0, 0)),
            pl.BlockSpec((CONV_S, NT, CG, C), lambda m: (m, 0, 0, 0)),
            pl.BlockSpec((NT, P, P), lambda m: (0, 0, 0)),
            pl.BlockSpec((1, C, P), lambda m: (0, 0, 0)),
            pl.BlockSpec((1, C, P), lambda m: (0, 0, 0)),
        ],
        out_specs=pl.BlockSpec((CONV_S, C, P), lambda m: (m, 0, 0)),
        out_shape=jax.ShapeDtypeStruct((K, C, P), F32),
    )(x3, dw4, tmats, g2v, b2v)


FFN_BK = 896


def _ffn1_kernel(x_ref, w_ref, b_ref, o_ref):
    k = pl.program_id(0)
    acc = jnp.dot(x_ref[...], w_ref[...], preferred_element_type=F32)

    @pl.when(k == 0)
    def _init():
        o_ref[...] = acc

    @pl.when(k > 0)
    def _acc():
        o_ref[...] = o_ref[...] + acc

    @pl.when(k == NB - 1)
    def _fin():
        o_ref[...] = jnp.maximum(o_ref[...] + b_ref[...], 0.0)


def _ffn1(xf, w1t, b1):
    return pl.pallas_call(
        _ffn1_kernel,
        grid=(NB,),
        in_specs=[
            pl.BlockSpec((K, FFN_BK), lambda k: (0, k)),
            pl.BlockSpec((FFN_BK, FFN), lambda k: (k, 0)),
            pl.BlockSpec((1, FFN), lambda k: (0, 0)),
        ],
        out_specs=pl.BlockSpec((K, FFN), lambda k: (0, 0)),
        out_shape=jax.ShapeDtypeStruct((K, FFN), F32),
    )(xf, w1t, b1)


def _ffn2_ln_kernel(h_ref, w_ref, b_ref, r_ref, g_ref, bb_ref, o_ref):
    y = (jnp.dot(h_ref[...], w_ref[...], preferred_element_type=F32)
         + b_ref[...] + r_ref[...])
    mu = jnp.mean(y, axis=-1, keepdims=True)
    yc = y - mu
    var = jnp.mean(yc * yc, axis=-1, keepdims=True)
    o_ref[...] = yc * jax.lax.rsqrt(var + 1e-5) * g_ref[...] + bb_ref[...]


def _ffn2_ln(h, w2t, b2, res, g3, b3):
    return pl.pallas_call(
        _ffn2_ln_kernel,
        grid=(MB,),
        in_specs=[
            pl.BlockSpec((BM, FFN), lambda m: (m, 0)),
            pl.BlockSpec((FFN, D), lambda m: (0, 0)),
            pl.BlockSpec((1, D), lambda m: (0, 0)),
            pl.BlockSpec((BM, D), lambda m: (m, 0)),
            pl.BlockSpec((1, D), lambda m: (0, 0)),
            pl.BlockSpec((1, D), lambda m: (0, 0)),
        ],
        out_specs=pl.BlockSpec((BM, D), lambda m: (m, 0)),
        out_shape=jax.ShapeDtypeStruct((K, D), F32),
    )(h, w2t, b2, res, g3, b3)


def _cost_kernel(rb_ref, cbt_ref, o_ref):
    rb = rb_ref[...]
    cbt = cbt_ref[...]
    ax0, ay0, ax1, ay1 = (rb[:, 0:1], rb[:, 1:2], rb[:, 2:3], rb[:, 3:4])
    bx0, by0, bx1, by1 = (cbt[0:1, :], cbt[1:2, :], cbt[2:3, :], cbt[3:4, :])
    area_a = (ax1 - ax0) * (ay1 - ay0)
    area_b = (bx1 - bx0) * (by1 - by0)
    wx = jnp.clip(jnp.minimum(ax1, bx1) - jnp.maximum(ax0, bx0), 0.0)
    wy = jnp.clip(jnp.minimum(ay1, by1) - jnp.maximum(ay0, by0), 0.0)
    inter = wx * wy
    iou = inter / (area_a + area_b - inter)
    dist = (jnp.abs((ax0 + ax1) / 2 - (bx0 + bx1) / 2)
            + jnp.abs((ay0 + ay1) / 2 - (by0 + by1) / 2))
    dn = dist / jnp.clip(jnp.max(dist), 1.0)
    o_ref[...] = -1.0 * iou + 0.5 * dn


def _cost(rb, cbt):
    return pl.pallas_call(
        _cost_kernel,
        in_specs=[
            pl.BlockSpec((M, 4), lambda: (0, 0)),
            pl.BlockSpec((4, M), lambda: (0, 0)),
        ],
        out_specs=pl.BlockSpec((M, M), lambda: (0, 0)),
        out_shape=jax.ShapeDtypeStruct((M, M), F32),
    )(rb, cbt)


def _shift_mats():
    t = np.zeros((NT, P, P), np.float32)
    for dy in range(KS):
        for dx in range(KS):
            for y in range(R):
                for x in range(R):
                    qy, qx = y + dy - 1, x + dx - 1
                    if 0 <= qy < R and 0 <= qx < R:
                        t[dy * KS + dx, qy * R + qx, y * R + x] = 1.0
    return jnp.asarray(t)


@jax.jit
def kernel(wrong_features, right_features, ref_boxes, curr_boxes, wq, bq, wk,
           bk, wv, bv, wo, bo, g1, b1, wgen_w, wgen_b, g2, b2, wf1, bf1, wf2,
           bf2, g3, b3):
    wf = wrong_features.reshape(K, D)
    rf = right_features.reshape(K, D)

    v = _mm_bias(rf, wv.T, bv.reshape(1, D))
    aor = _mm_bias_res(v, wo.T, bo.reshape(1, D), wf)
    x3, gap = _ln_gap(aor.reshape(K, C, P), g1.reshape(1, C, P),
                      b1.reshape(1, C, P))

    wgp = wgen_w.T.reshape(C, C, CG, NT).transpose(0, 3, 2, 1).reshape(C, -1)
    bgp = wgen_b.reshape(C, CG, NT).transpose(2, 1, 0).reshape(1, -1)
    dw = _gen(gap, wgp, bgp)

    dc3 = _conv_ln(x3, dw.reshape(K, NT, CG, C), _shift_mats(),
                   g2.reshape(1, C, P), b2.reshape(1, C, P))

    xf = dc3.reshape(K, D)
    h = _ffn1(xf, wf1.T, bf1.reshape(1, FFN))
    out = _ffn2_ln(h, wf2.T, bf2.reshape(1, D), xf, g3.reshape(1, D),
                   b3.reshape(1, D))

    cost = _cost(ref_boxes, curr_boxes.T)
    return out.reshape(K, C, R, R), cost


# speedup vs baseline: 46.2396x; 1.0134x over previous
import functools

import jax
import jax.numpy as jnp
import numpy as np
from jax.experimental import pallas as pl

K, C, R, NH, FFN, GROUPS, KS, M = 1024, 128, 7, 8, 1024, 4, 3, 64
D = C * R * R
P = R * R
CG = C // GROUPS
NT = KS * KS
BN = 896
NB = D // BN
BM = 128
MB = K // BM
F32 = jnp.float32


def _dott(x, w):
    return jax.lax.dot_general(x, w, (((1,), (1,)), ((), ())),
                               preferred_element_type=F32)


def _mm_bias_kernel(x_ref, w_ref, b_ref, o_ref):
    o_ref[...] = _dott(x_ref[...], w_ref[...]) + b_ref[...]


def _mm_bias(x, w, b):
    n_blocks = w.shape[0] // BN
    return pl.pallas_call(
        _mm_bias_kernel,
        grid=(n_blocks, MB),
        in_specs=[
            pl.BlockSpec((BM, x.shape[1]), lambda n, m: (m, 0)),
            pl.BlockSpec((BN, w.shape[1]), lambda n, m: (n, 0)),
            pl.BlockSpec((1, BN), lambda n, m: (0, n)),
        ],
        out_specs=pl.BlockSpec((BM, BN), lambda n, m: (m, n)),
        out_shape=jax.ShapeDtypeStruct((x.shape[0], w.shape[0]), F32),
    )(x, w, b)


def _mm_bias_res_kernel(x_ref, w_ref, b_ref, r_ref, o_ref):
    o_ref[...] = _dott(x_ref[...], w_ref[...]) + b_ref[...] + r_ref[...]


def _mm_bias_res(x, w, b, res):
    n_blocks = w.shape[0] // BN
    return pl.pallas_call(
        _mm_bias_res_kernel,
        grid=(n_blocks, MB),
        in_specs=[
            pl.BlockSpec((BM, x.shape[1]), lambda n, m: (m, 0)),
            pl.BlockSpec((BN, w.shape[1]), lambda n, m: (n, 0)),
            pl.BlockSpec((1, BN), lambda n, m: (0, n)),
            pl.BlockSpec((BM, BN), lambda n, m: (m, n)),
        ],
        out_specs=pl.BlockSpec((BM, BN), lambda n, m: (m, n)),
        out_shape=jax.ShapeDtypeStruct((x.shape[0], w.shape[0]), F32),
    )(x, w, b, res)


def _ln_gap_kernel(x_ref, g_ref, b_ref, y_ref, gap_ref):
    x = x_ref[...]
    mu = jnp.mean(x, axis=(1, 2), keepdims=True)
    xc = x - mu
    var = jnp.mean(xc * xc, axis=(1, 2), keepdims=True)
    y = xc * jax.lax.rsqrt(var + 1e-5) * g_ref[...] + b_ref[...]
    y_ref[...] = y
    gap_ref[...] = jnp.mean(y, axis=2)


def _ln_gap(x3, g, b):
    return pl.pallas_call(
        _ln_gap_kernel,
        grid=(MB,),
        in_specs=[
            pl.BlockSpec((BM, C, P), lambda m: (m, 0, 0)),
            pl.BlockSpec((1, C, P), lambda m: (0, 0, 0)),
            pl.BlockSpec((1, C, P), lambda m: (0, 0, 0)),
        ],
        out_specs=[
            pl.BlockSpec((BM, C, P), lambda m: (m, 0, 0)),
            pl.BlockSpec((BM, C), lambda m: (m, 0)),
        ],
        out_shape=[
            jax.ShapeDtypeStruct((K, C, P), F32),
            jax.ShapeDtypeStruct((K, C), F32),
        ],
    )(x3, g, b)


DW_BN = 2304


def _gen_kernel(g_ref, w_ref, b_ref, o_ref):
    o_ref[...] = jnp.dot(g_ref[...], w_ref[...],
                         preferred_element_type=F32) + b_ref[...]


def _gen(gap, wgp, bgp):
    n_blocks = wgp.shape[1] // DW_BN
    return pl.pallas_call(
        _gen_kernel,
        grid=(n_blocks,),
        in_specs=[
            pl.BlockSpec((K, C), lambda n: (0, 0)),
            pl.BlockSpec((C, DW_BN), lambda n: (0, n)),
            pl.BlockSpec((1, DW_BN), lambda n: (0, n)),
        ],
        out_specs=pl.BlockSpec((K, DW_BN), lambda n: (0, n)),
        out_shape=jax.ShapeDtypeStruct((K, wgp.shape[1]), F32),
    )(gap, wgp, bgp)


CONV_S = 8


def _conv_ln_kernel(x_ref, dw_ref, t_ref, g_ref, b_ref, o_ref):
    S = CONV_S
    x = x_ref[...]
    xs2 = x.reshape(S * C, P)
    shs = [jnp.dot(xs2, t_ref[t], preferred_element_type=F32).reshape(S, C, P)
           for t in range(NT)]
    patches = jnp.stack(shs, axis=1).reshape(S, NT * C, P)
    dwv = dw_ref[...]
    wtile = jnp.broadcast_to(dwv[:, :, None, :, :],
                             (S, NT, GROUPS, CG, C)).reshape(S, NT, C, C)
    ci = jax.lax.broadcasted_iota(jnp.int32, (C, C), 0)
    co = jax.lax.broadcasted_iota(jnp.int32, (C, C), 1)
    mask = (ci // CG == co // CG).astype(F32)
    wd = (wtile * mask).reshape(S, NT * C, C)
    outs = []
    for s in range(S):
        outs.append(jax.lax.dot_general(
            wd[s], patches[s], (((0,), (0,)), ((), ())),
            preferred_element_type=F32))
    dc = jnp.stack(outs, axis=0)
    y = dc + x
    mu = jnp.mean(y, axis=(1, 2), keepdims=True)
    yc = y - mu
    var = jnp.mean(yc * yc, axis=(1, 2), keepdims=True)
    o_ref[...] = yc * jax.lax.rsqrt(var + 1e-5) * g_ref[...] + b_ref[...]


def _conv_ln(x3, dw4, tmats, g2v, b2v):
    return pl.pallas_call(
        _conv_ln_kernel,
        grid=(K // CONV_S,),
        in_specs=[
            pl.BlockSpec((CONV_S, C, P), lambda m: (m, 0, 0)),
            pl.BlockSpec((CONV_S, NT, CG, C), lambda m: (m, 0, 0, 0)),
            pl.BlockSpec((NT, P, P), lambda m: (0, 0, 0)),
            pl.BlockSpec((1, C, P), lambda m: (0, 0, 0)),
            pl.BlockSpec((1, C, P), lambda m: (0, 0, 0)),
        ],
        out_specs=pl.BlockSpec((CONV_S, C, P), lambda m: (m, 0, 0)),
        out_shape=jax.ShapeDtypeStruct((K, C, P), F32),
    )(x3, dw4, tmats, g2v, b2v)


FFN_BK = 896


def _ffn1_kernel(x_ref, w_ref, b_ref, o_ref):
    k = pl.program_id(0)
    acc = _dott(x_ref[...], w_ref[...])

    @pl.when(k == 0)
    def _init():
        o_ref[...] = acc

    @pl.when(k > 0)
    def _acc():
        o_ref[...] = o_ref[...] + acc

    @pl.when(k == NB - 1)
    def _fin():
        o_ref[...] = jnp.maximum(o_ref[...] + b_ref[...], 0.0)


def _ffn1(xf, w1, b1):
    return pl.pallas_call(
        _ffn1_kernel,
        grid=(NB,),
        in_specs=[
            pl.BlockSpec((K, FFN_BK), lambda k: (0, k)),
            pl.BlockSpec((FFN, FFN_BK), lambda k: (0, k)),
            pl.BlockSpec((1, FFN), lambda k: (0, 0)),
        ],
        out_specs=pl.BlockSpec((K, FFN), lambda k: (0, 0)),
        out_shape=jax.ShapeDtypeStruct((K, FFN), F32),
    )(xf, w1, b1)


def _ffn2_ln_kernel(h_ref, w_ref, b_ref, r_ref, g_ref, bb_ref, o_ref):
    y = _dott(h_ref[...], w_ref[...]) + b_ref[...] + r_ref[...]
    mu = jnp.mean(y, axis=-1, keepdims=True)
    yc = y - mu
    var = jnp.mean(yc * yc, axis=-1, keepdims=True)
    o_ref[...] = yc * jax.lax.rsqrt(var + 1e-5) * g_ref[...] + bb_ref[...]


def _ffn2_ln(h, w2, b2, res, g3, b3):
    return pl.pallas_call(
        _ffn2_ln_kernel,
        grid=(MB,),
        in_specs=[
            pl.BlockSpec((BM, FFN), lambda m: (m, 0)),
            pl.BlockSpec((D, FFN), lambda m: (0, 0)),
            pl.BlockSpec((1, D), lambda m: (0, 0)),
            pl.BlockSpec((BM, D), lambda m: (m, 0)),
            pl.BlockSpec((1, D), lambda m: (0, 0)),
            pl.BlockSpec((1, D), lambda m: (0, 0)),
        ],
        out_specs=pl.BlockSpec((BM, D), lambda m: (m, 0)),
        out_shape=jax.ShapeDtypeStruct((K, D), F32),
    )(h, w2, b2, res, g3, b3)


def _cost_kernel(rb_ref, cbt_ref, o_ref):
    rb = rb_ref[...]
    cbt = cbt_ref[...]
    ax0, ay0, ax1, ay1 = (rb[:, 0:1], rb[:, 1:2], rb[:, 2:3], rb[:, 3:4])
    bx0, by0, bx1, by1 = (cbt[0:1, :], cbt[1:2, :], cbt[2:3, :], cbt[3:4, :])
    area_a = (ax1 - ax0) * (ay1 - ay0)
    area_b = (bx1 - bx0) * (by1 - by0)
    wx = jnp.clip(jnp.minimum(ax1, bx1) - jnp.maximum(ax0, bx0), 0.0)
    wy = jnp.clip(jnp.minimum(ay1, by1) - jnp.maximum(ay0, by0), 0.0)
    inter = wx * wy
    iou = inter / (area_a + area_b - inter)
    dist = (jnp.abs((ax0 + ax1) / 2 - (bx0 + bx1) / 2)
            + jnp.abs((ay0 + ay1) / 2 - (by0 + by1) / 2))
    dn = dist / jnp.clip(jnp.max(dist), 1.0)
    o_ref[...] = -1.0 * iou + 0.5 * dn


def _cost(rb, cbt):
    return pl.pallas_call(
        _cost_kernel,
        in_specs=[
            pl.BlockSpec((M, 4), lambda: (0, 0)),
            pl.BlockSpec((4, M), lambda: (0, 0)),
        ],
        out_specs=pl.BlockSpec((M, M), lambda: (0, 0)),
        out_shape=jax.ShapeDtypeStruct((M, M), F32),
    )(rb, cbt)


def _shift_mats():
    t = np.zeros((NT, P, P), np.float32)
    for dy in range(KS):
        for dx in range(KS):
            for y in range(R):
                for x in range(R):
                    qy, qx = y + dy - 1, x + dx - 1
                    if 0 <= qy < R and 0 <= qx < R:
                        t[dy * KS + dx, qy * R + qx, y * R + x] = 1.0
    return jnp.asarray(t)


@jax.jit
def kernel(wrong_features, right_features, ref_boxes, curr_boxes, wq, bq, wk,
           bk, wv, bv, wo, bo, g1, b1, wgen_w, wgen_b, g2, b2, wf1, bf1, wf2,
           bf2, g3, b3):
    wf = wrong_features.reshape(K, D)
    rf = right_features.reshape(K, D)

    v = _mm_bias(rf, wv, bv.reshape(1, D))
    aor = _mm_bias_res(v, wo, bo.reshape(1, D), wf)
    x3, gap = _ln_gap(aor.reshape(K, C, P), g1.reshape(1, C, P),
                      b1.reshape(1, C, P))

    wgp = wgen_w.T.reshape(C, C, CG, NT).transpose(0, 3, 2, 1).reshape(C, -1)
    bgp = wgen_b.reshape(C, CG, NT).transpose(2, 1, 0).reshape(1, -1)
    dw = _gen(gap, wgp, bgp)

    dc3 = _conv_ln(x3, dw.reshape(K, NT, CG, C), _shift_mats(),
                   g2.reshape(1, C, P), b2.reshape(1, C, P))

    xf = dc3.reshape(K, D)
    h = _ffn1(xf, wf1, bf1.reshape(1, FFN))
    out = _ffn2_ln(h, wf2, bf2.reshape(1, D), xf, g3.reshape(1, D),
                   b3.reshape(1, D))

    cost = _cost(ref_boxes, curr_boxes.T)
    return out.reshape(K, C, R, R), cost


# speedup vs baseline: 46.9237x; 1.0148x over previous
import functools

import jax
import jax.numpy as jnp
import numpy as np
from jax.experimental import pallas as pl

K, C, R, NH, FFN, GROUPS, KS, M = 1024, 128, 7, 8, 1024, 4, 3, 64
D = C * R * R
P = R * R
CG = C // GROUPS
NT = KS * KS
BN = 896
NB = D // BN
BM = 128
MB = K // BM
F32 = jnp.float32


def _dott(x, w):
    return jax.lax.dot_general(x, w, (((1,), (1,)), ((), ())),
                               preferred_element_type=F32)


def _mm_bias_kernel(x_ref, w_ref, b_ref, o_ref):
    o_ref[...] = _dott(x_ref[...], w_ref[...]) + b_ref[...]


def _mm_bias(x, w, b):
    n_blocks = w.shape[0] // BN
    return pl.pallas_call(
        _mm_bias_kernel,
        grid=(n_blocks, MB),
        in_specs=[
            pl.BlockSpec((BM, x.shape[1]), lambda n, m: (m, 0)),
            pl.BlockSpec((BN, w.shape[1]), lambda n, m: (n, 0)),
            pl.BlockSpec((1, BN), lambda n, m: (0, n)),
        ],
        out_specs=pl.BlockSpec((BM, BN), lambda n, m: (m, n)),
        out_shape=jax.ShapeDtypeStruct((x.shape[0], w.shape[0]), F32),
    )(x, w, b)


def _mm_bias_res_kernel(x_ref, w_ref, b_ref, r_ref, o_ref):
    o_ref[...] = _dott(x_ref[...], w_ref[...]) + b_ref[...] + r_ref[...]


def _mm_bias_res(x, w, b, res):
    n_blocks = w.shape[0] // BN
    return pl.pallas_call(
        _mm_bias_res_kernel,
        grid=(n_blocks, MB),
        in_specs=[
            pl.BlockSpec((BM, x.shape[1]), lambda n, m: (m, 0)),
            pl.BlockSpec((BN, w.shape[1]), lambda n, m: (n, 0)),
            pl.BlockSpec((1, BN), lambda n, m: (0, n)),
            pl.BlockSpec((BM, BN), lambda n, m: (m, n)),
        ],
        out_specs=pl.BlockSpec((BM, BN), lambda n, m: (m, n)),
        out_shape=jax.ShapeDtypeStruct((x.shape[0], w.shape[0]), F32),
    )(x, w, b, res)


def _ln_gap_kernel(x_ref, g_ref, b_ref, y_ref, gap_ref):
    x = x_ref[...]
    mu = jnp.mean(x, axis=(1, 2), keepdims=True)
    xc = x - mu
    var = jnp.mean(xc * xc, axis=(1, 2), keepdims=True)
    y = xc * jax.lax.rsqrt(var + 1e-5) * g_ref[...] + b_ref[...]
    y_ref[...] = y
    gap_ref[...] = jnp.mean(y, axis=2)


def _ln_gap(x3, g, b):
    return pl.pallas_call(
        _ln_gap_kernel,
        grid=(MB,),
        in_specs=[
            pl.BlockSpec((BM, C, P), lambda m: (m, 0, 0)),
            pl.BlockSpec((1, C, P), lambda m: (0, 0, 0)),
            pl.BlockSpec((1, C, P), lambda m: (0, 0, 0)),
        ],
        out_specs=[
            pl.BlockSpec((BM, C, P), lambda m: (m, 0, 0)),
            pl.BlockSpec((BM, C), lambda m: (m, 0)),
        ],
        out_shape=[
            jax.ShapeDtypeStruct((K, C, P), F32),
            jax.ShapeDtypeStruct((K, C), F32),
        ],
    )(x3, g, b)


DW_BN = 2304


def _gen_kernel(g_ref, w_ref, b_ref, o_ref):
    o_ref[...] = jnp.dot(g_ref[...], w_ref[...],
                         preferred_element_type=F32) + b_ref[...]


def _gen(gap, wgp, bgp):
    n_blocks = wgp.shape[1] // DW_BN
    return pl.pallas_call(
        _gen_kernel,
        grid=(n_blocks,),
        in_specs=[
            pl.BlockSpec((K, C), lambda n: (0, 0)),
            pl.BlockSpec((C, DW_BN), lambda n: (0, n)),
            pl.BlockSpec((1, DW_BN), lambda n: (0, n)),
        ],
        out_specs=pl.BlockSpec((K, DW_BN), lambda n: (0, n)),
        out_shape=jax.ShapeDtypeStruct((K, wgp.shape[1]), F32),
    )(gap, wgp, bgp)


CONV_S = 16


def _conv_ln_kernel(x_ref, dw_ref, t_ref, g_ref, b_ref, o_ref):
    S = CONV_S
    x = x_ref[...]
    xs2 = x.reshape(S * C, P)
    shs = [jnp.dot(xs2, t_ref[t], preferred_element_type=F32).reshape(S, C, P)
           for t in range(NT)]
    patches = jnp.stack(shs, axis=1).reshape(S, NT * C, P)
    dwv = dw_ref[...]
    wtile = jnp.broadcast_to(dwv[:, :, None, :, :],
                             (S, NT, GROUPS, CG, C)).reshape(S, NT, C, C)
    ci = jax.lax.broadcasted_iota(jnp.int32, (C, C), 0)
    co = jax.lax.broadcasted_iota(jnp.int32, (C, C), 1)
    mask = (ci // CG == co // CG).astype(F32)
    wd = (wtile * mask).reshape(S, NT * C, C)
    outs = []
    for s in range(S):
        outs.append(jax.lax.dot_general(
            wd[s], patches[s], (((0,), (0,)), ((), ())),
            preferred_element_type=F32))
    dc = jnp.stack(outs, axis=0)
    y = dc + x
    mu = jnp.mean(y, axis=(1, 2), keepdims=True)
    yc = y - mu
    var = jnp.mean(yc * yc, axis=(1, 2), keepdims=True)
    o_ref[...] = yc * jax.lax.rsqrt(var + 1e-5) * g_ref[...] + b_ref[...]


def _conv_ln(x3, dw4, tmats, g2v, b2v):
    return pl.pallas_call(
        _conv_ln_kernel,
        grid=(K // CONV_S,),
        in_specs=[
            pl.BlockSpec((CONV_S, C, P), lambda m: (m, 0, 0)),
            pl.BlockSpec((CONV_S, NT, CG, C), lambda m: (m, 0, 0, 0)),
            pl.BlockSpec((NT, P, P), lambda m: (0, 0, 0)),
            pl.BlockSpec((1, C, P), lambda m: (0, 0, 0)),
            pl.BlockSpec((1, C, P), lambda m: (0, 0, 0)),
        ],
        out_specs=pl.BlockSpec((CONV_S, C, P), lambda m: (m, 0, 0)),
        out_shape=jax.ShapeDtypeStruct((K, C, P), F32),
    )(x3, dw4, tmats, g2v, b2v)


FFN_BK = 896


def _ffn1_kernel(x_ref, w_ref, b_ref, o_ref):
    k = pl.program_id(0)
    acc = _dott(x_ref[...], w_ref[...])

    @pl.when(k == 0)
    def _init():
        o_ref[...] = acc

    @pl.when(k > 0)
    def _acc():
        o_ref[...] = o_ref[...] + acc

    @pl.when(k == NB - 1)
    def _fin():
        o_ref[...] = jnp.maximum(o_ref[...] + b_ref[...], 0.0)


def _ffn1(xf, w1, b1):
    return pl.pallas_call(
        _ffn1_kernel,
        grid=(NB,),
        in_specs=[
            pl.BlockSpec((K, FFN_BK), lambda k: (0, k)),
            pl.BlockSpec((FFN, FFN_BK), lambda k: (0, k)),
            pl.BlockSpec((1, FFN), lambda k: (0, 0)),
        ],
        out_specs=pl.BlockSpec((K, FFN), lambda k: (0, 0)),
        out_shape=jax.ShapeDtypeStruct((K, FFN), F32),
    )(xf, w1, b1)


def _ffn2_ln_kernel(h_ref, w_ref, b_ref, r_ref, g_ref, bb_ref, o_ref):
    y = _dott(h_ref[...], w_ref[...]) + b_ref[...] + r_ref[...]
    mu = jnp.mean(y, axis=-1, keepdims=True)
    yc = y - mu
    var = jnp.mean(yc * yc, axis=-1, keepdims=True)
    o_ref[...] = yc * jax.lax.rsqrt(var + 1e-5) * g_ref[...] + bb_ref[...]


def _ffn2_ln(h, w2, b2, res, g3, b3):
    return pl.pallas_call(
        _ffn2_ln_kernel,
        grid=(MB,),
        in_specs=[
            pl.BlockSpec((BM, FFN), lambda m: (m, 0)),
            pl.BlockSpec((D, FFN), lambda m: (0, 0)),
            pl.BlockSpec((1, D), lambda m: (0, 0)),
            pl.BlockSpec((BM, D), lambda m: (m, 0)),
            pl.BlockSpec((1, D), lambda m: (0, 0)),
            pl.BlockSpec((1, D), lambda m: (0, 0)),
        ],
        out_specs=pl.BlockSpec((BM, D), lambda m: (m, 0)),
        out_shape=jax.ShapeDtypeStruct((K, D), F32),
    )(h, w2, b2, res, g3, b3)


def _cost_kernel(rb_ref, cbt_ref, o_ref):
    rb = rb_ref[...]
    cbt = cbt_ref[...]
    ax0, ay0, ax1, ay1 = (rb[:, 0:1], rb[:, 1:2], rb[:, 2:3], rb[:, 3:4])
    bx0, by0, bx1, by1 = (cbt[0:1, :], cbt[1:2, :], cbt[2:3, :], cbt[3:4, :])
    area_a = (ax1 - ax0) * (ay1 - ay0)
    area_b = (bx1 - bx0) * (by1 - by0)
    wx = jnp.clip(jnp.minimum(ax1, bx1) - jnp.maximum(ax0, bx0), 0.0)
    wy = jnp.clip(jnp.minimum(ay1, by1) - jnp.maximum(ay0, by0), 0.0)
    inter = wx * wy
    iou = inter / (area_a + area_b - inter)
    dist = (jnp.abs((ax0 + ax1) / 2 - (bx0 + bx1) / 2)
            + jnp.abs((ay0 + ay1) / 2 - (by0 + by1) / 2))
    dn = dist / jnp.clip(jnp.max(dist), 1.0)
    o_ref[...] = -1.0 * iou + 0.5 * dn


def _cost(rb, cbt):
    return pl.pallas_call(
        _cost_kernel,
        in_specs=[
            pl.BlockSpec((M, 4), lambda: (0, 0)),
            pl.BlockSpec((4, M), lambda: (0, 0)),
        ],
        out_specs=pl.BlockSpec((M, M), lambda: (0, 0)),
        out_shape=jax.ShapeDtypeStruct((M, M), F32),
    )(rb, cbt)


def _shift_mats():
    t = np.zeros((NT, P, P), np.float32)
    for dy in range(KS):
        for dx in range(KS):
            for y in range(R):
                for x in range(R):
                    qy, qx = y + dy - 1, x + dx - 1
                    if 0 <= qy < R and 0 <= qx < R:
                        t[dy * KS + dx, qy * R + qx, y * R + x] = 1.0
    return jnp.asarray(t)


@jax.jit
def kernel(wrong_features, right_features, ref_boxes, curr_boxes, wq, bq, wk,
           bk, wv, bv, wo, bo, g1, b1, wgen_w, wgen_b, g2, b2, wf1, bf1, wf2,
           bf2, g3, b3):
    wf = wrong_features.reshape(K, D)
    rf = right_features.reshape(K, D)

    v = _mm_bias(rf, wv, bv.reshape(1, D))
    aor = _mm_bias_res(v, wo, bo.reshape(1, D), wf)
    x3, gap = _ln_gap(aor.reshape(K, C, P), g1.reshape(1, C, P),
                      b1.reshape(1, C, P))

    wgp = wgen_w.T.reshape(C, C, CG, NT).transpose(0, 3, 2, 1).reshape(C, -1)
    bgp = wgen_b.reshape(C, CG, NT).transpose(2, 1, 0).reshape(1, -1)
    dw = _gen(gap, wgp, bgp)

    dc3 = _conv_ln(x3, dw.reshape(K, NT, CG, C), _shift_mats(),
                   g2.reshape(1, C, P), b2.reshape(1, C, P))

    xf = dc3.reshape(K, D)
    h = _ffn1(xf, wf1, bf1.reshape(1, FFN))
    out = _ffn2_ln(h, wf2, bf2.reshape(1, D), xf, g3.reshape(1, D),
                   b3.reshape(1, D))

    cost = _cost(ref_boxes, curr_boxes.T)
    return out.reshape(K, C, R, R), cost


# speedup vs baseline: 46.9730x; 1.0011x over previous
import functools

import jax
import jax.numpy as jnp
import numpy as np
from jax.experimental import pallas as pl

K, C, R, NH, FFN, GROUPS, KS, M = 1024, 128, 7, 8, 1024, 4, 3, 64
D = C * R * R
P = R * R
CG = C // GROUPS
NT = KS * KS
BN = 896
NB = D // BN
BM = 128
MB = K // BM
F32 = jnp.float32


BF16 = jnp.bfloat16


def _dott(x, w):
    return jax.lax.dot_general(x.astype(BF16), w.astype(BF16),
                               (((1,), (1,)), ((), ())),
                               preferred_element_type=F32)


def _mm_bias_nt_kernel(x_ref, w_ref, b_ref, o_ref):
    o_ref[...] = jnp.dot(x_ref[...], w_ref[...],
                         preferred_element_type=F32) + b_ref[...]


def _mm_bias_nt(x, w, b):
    n_blocks = w.shape[1] // BN
    return pl.pallas_call(
        _mm_bias_nt_kernel,
        grid=(n_blocks, MB),
        in_specs=[
            pl.BlockSpec((BM, x.shape[1]), lambda n, m: (m, 0)),
            pl.BlockSpec((w.shape[0], BN), lambda n, m: (0, n)),
            pl.BlockSpec((1, BN), lambda n, m: (0, n)),
        ],
        out_specs=pl.BlockSpec((BM, BN), lambda n, m: (m, n)),
        out_shape=jax.ShapeDtypeStruct((x.shape[0], w.shape[1]), F32),
    )(x, w, b)


def _mm_bias_kernel(x_ref, w_ref, b_ref, o_ref):
    o_ref[...] = _dott(x_ref[...], w_ref[...]) + b_ref[...]


def _mm_bias(x, w, b):
    n_blocks = w.shape[0] // BN
    return pl.pallas_call(
        _mm_bias_kernel,
        grid=(n_blocks, MB),
        in_specs=[
            pl.BlockSpec((BM, x.shape[1]), lambda n, m: (m, 0)),
            pl.BlockSpec((BN, w.shape[1]), lambda n, m: (n, 0)),
            pl.BlockSpec((1, BN), lambda n, m: (0, n)),
        ],
        out_specs=pl.BlockSpec((BM, BN), lambda n, m: (m, n)),
        out_shape=jax.ShapeDtypeStruct((x.shape[0], w.shape[0]), F32),
    )(x, w, b)


def _mm_bias_res_kernel(x_ref, w_ref, b_ref, r_ref, o_ref):
    o_ref[...] = _dott(x_ref[...], w_ref[...]) + b_ref[...] + r_ref[...]


def _mm_bias_res(x, w, b, res):
    n_blocks = w.shape[0] // BN
    return pl.pallas_call(
        _mm_bias_res_kernel,
        grid=(n_blocks, MB),
        in_specs=[
            pl.BlockSpec((BM, x.shape[1]), lambda n, m: (m, 0)),
            pl.BlockSpec((BN, w.shape[1]), lambda n, m: (n, 0)),
            pl.BlockSpec((1, BN), lambda n, m: (0, n)),
            pl.BlockSpec((BM, BN), lambda n, m: (m, n)),
        ],
        out_specs=pl.BlockSpec((BM, BN), lambda n, m: (m, n)),
        out_shape=jax.ShapeDtypeStruct((x.shape[0], w.shape[0]), F32),
    )(x, w, b, res)


def _ln_gap_kernel(x_ref, g_ref, b_ref, y_ref, gap_ref):
    x = x_ref[...]
    mu = jnp.mean(x, axis=(1, 2), keepdims=True)
    xc = x - mu
    var = jnp.mean(xc * xc, axis=(1, 2), keepdims=True)
    y = xc * jax.lax.rsqrt(var + 1e-5) * g_ref[...] + b_ref[...]
    y_ref[...] = y
    gap_ref[...] = jnp.mean(y, axis=2)


def _ln_gap(x3, g, b):
    return pl.pallas_call(
        _ln_gap_kernel,
        grid=(MB,),
        in_specs=[
            pl.BlockSpec((BM, C, P), lambda m: (m, 0, 0)),
            pl.BlockSpec((1, C, P), lambda m: (0, 0, 0)),
            pl.BlockSpec((1, C, P), lambda m: (0, 0, 0)),
        ],
        out_specs=[
            pl.BlockSpec((BM, C, P), lambda m: (m, 0, 0)),
            pl.BlockSpec((BM, C), lambda m: (m, 0)),
        ],
        out_shape=[
            jax.ShapeDtypeStruct((K, C, P), F32),
            jax.ShapeDtypeStruct((K, C), F32),
        ],
    )(x3, g, b)


DW_BN = 2304


def _gen_kernel(g_ref, w_ref, b_ref, o_ref):
    o_ref[...] = jnp.dot(g_ref[...].astype(BF16), w_ref[...].astype(BF16),
                         preferred_element_type=F32) + b_ref[...]


def _gen(gap, wgp, bgp):
    n_blocks = wgp.shape[1] // DW_BN
    return pl.pallas_call(
        _gen_kernel,
        grid=(n_blocks,),
        in_specs=[
            pl.BlockSpec((K, C), lambda n: (0, 0)),
            pl.BlockSpec((C, DW_BN), lambda n: (0, n)),
            pl.BlockSpec((1, DW_BN), lambda n: (0, n)),
        ],
        out_specs=pl.BlockSpec((K, DW_BN), lambda n: (0, n)),
        out_shape=jax.ShapeDtypeStruct((K, wgp.shape[1]), F32),
    )(gap, wgp, bgp)


CONV_S = 16


def _conv_ln_kernel(x_ref, dw_ref, t_ref, g_ref, b_ref, o_ref):
    S = CONV_S
    x = x_ref[...]
    xs2 = x.reshape(S * C, P).astype(BF16)
    shs = [jnp.dot(xs2, t_ref[t].astype(BF16),
                   preferred_element_type=F32).astype(BF16).reshape(S, C, P)
           for t in range(NT)]
    patches = jnp.stack(shs, axis=1).reshape(S, NT * C, P)
    dwv = dw_ref[...]
    wtile = jnp.broadcast_to(dwv[:, :, None, :, :],
                             (S, NT, GROUPS, CG, C)).reshape(S, NT, C, C)
    ci = jax.lax.broadcasted_iota(jnp.int32, (C, C), 0)
    co = jax.lax.broadcasted_iota(jnp.int32, (C, C), 1)
    mask = (ci // CG == co // CG).astype(F32)
    wd = (wtile * mask).reshape(S, NT * C, C).astype(BF16)
    outs = []
    for s in range(S):
        outs.append(jax.lax.dot_general(
            wd[s], patches[s], (((0,), (0,)), ((), ())),
            preferred_element_type=F32))
    dc = jnp.stack(outs, axis=0)
    y = dc + x
    mu = jnp.mean(y, axis=(1, 2), keepdims=True)
    yc = y - mu
    var = jnp.mean(yc * yc, axis=(1, 2), keepdims=True)
    o_ref[...] = yc * jax.lax.rsqrt(var + 1e-5) * g_ref[...] + b_ref[...]


def _conv_ln(x3, dw4, tmats, g2v, b2v):
    return pl.pallas_call(
        _conv_ln_kernel,
        grid=(K // CONV_S,),
        in_specs=[
            pl.BlockSpec((CONV_S, C, P), lambda m: (m, 0, 0)),
            pl.BlockSpec((CONV_S, NT, CG, C), lambda m: (m, 0, 0, 0)),
            pl.BlockSpec((NT, P, P), lambda m: (0, 0, 0)),
            pl.BlockSpec((1, C, P), lambda m: (0, 0, 0)),
            pl.BlockSpec((1, C, P), lambda m: (0, 0, 0)),
        ],
        out_specs=pl.BlockSpec((CONV_S, C, P), lambda m: (m, 0, 0)),
        out_shape=jax.ShapeDtypeStruct((K, C, P), F32),
    )(x3, dw4, tmats, g2v, b2v)


FFN_BK = 896


def _ffn1_kernel(x_ref, w_ref, b_ref, o_ref):
    k = pl.program_id(0)
    acc = _dott(x_ref[...], w_ref[...])

    @pl.when(k == 0)
    def _init():
        o_ref[...] = acc

    @pl.when(k > 0)
    def _acc():
        o_ref[...] = o_ref[...] + acc

    @pl.when(k == NB - 1)
    def _fin():
        o_ref[...] = jnp.maximum(o_ref[...] + b_ref[...], 0.0)


def _ffn1(xf, w1, b1):
    return pl.pallas_call(
        _ffn1_kernel,
        grid=(NB,),
        in_specs=[
            pl.BlockSpec((K, FFN_BK), lambda k: (0, k)),
            pl.BlockSpec((FFN, FFN_BK), lambda k: (0, k)),
            pl.BlockSpec((1, FFN), lambda k: (0, 0)),
        ],
        out_specs=pl.BlockSpec((K, FFN), lambda k: (0, 0)),
        out_shape=jax.ShapeDtypeStruct((K, FFN), F32),
    )(xf, w1, b1)


def _ffn2_ln_kernel(h_ref, w_ref, b_ref, r_ref, g_ref, bb_ref, o_ref):
    y = _dott(h_ref[...], w_ref[...]) + b_ref[...] + r_ref[...]
    mu = jnp.mean(y, axis=-1, keepdims=True)
    yc = y - mu
    var = jnp.mean(yc * yc, axis=-1, keepdims=True)
    o_ref[...] = yc * jax.lax.rsqrt(var + 1e-5) * g_ref[...] + bb_ref[...]


def _ffn2_ln(h, w2, b2, res, g3, b3):
    return pl.pallas_call(
        _ffn2_ln_kernel,
        grid=(MB,),
        in_specs=[
            pl.BlockSpec((BM, FFN), lambda m: (m, 0)),
            pl.BlockSpec((D, FFN), lambda m: (0, 0)),
            pl.BlockSpec((1, D), lambda m: (0, 0)),
            pl.BlockSpec((BM, D), lambda m: (m, 0)),
            pl.BlockSpec((1, D), lambda m: (0, 0)),
            pl.BlockSpec((1, D), lambda m: (0, 0)),
        ],
        out_specs=pl.BlockSpec((BM, D), lambda m: (m, 0)),
        out_shape=jax.ShapeDtypeStruct((K, D), F32),
    )(h, w2, b2, res, g3, b3)


def _cost_kernel(rb_ref, cbt_ref, o_ref):
    rb = rb_ref[...]
    cbt = cbt_ref[...]
    ax0, ay0, ax1, ay1 = (rb[:, 0:1], rb[:, 1:2], rb[:, 2:3], rb[:, 3:4])
    bx0, by0, bx1, by1 = (cbt[0:1, :], cbt[1:2, :], cbt[2:3, :], cbt[3:4, :])
    area_a = (ax1 - ax0) * (ay1 - ay0)
    area_b = (bx1 - bx0) * (by1 - by0)
    wx = jnp.clip(jnp.minimum(ax1, bx1) - jnp.maximum(ax0, bx0), 0.0)
    wy = jnp.clip(jnp.minimum(ay1, by1) - jnp.maximum(ay0, by0), 0.0)
    inter = wx * wy
    iou = inter / (area_a + area_b - inter)
    dist = (jnp.abs((ax0 + ax1) / 2 - (bx0 + bx1) / 2)
            + jnp.abs((ay0 + ay1) / 2 - (by0 + by1) / 2))
    dn = dist / jnp.clip(jnp.max(dist), 1.0)
    o_ref[...] = -1.0 * iou + 0.5 * dn


def _cost(rb, cbt):
    return pl.pallas_call(
        _cost_kernel,
        in_specs=[
            pl.BlockSpec((M, 4), lambda: (0, 0)),
            pl.BlockSpec((4, M), lambda: (0, 0)),
        ],
        out_specs=pl.BlockSpec((M, M), lambda: (0, 0)),
        out_shape=jax.ShapeDtypeStruct((M, M), F32),
    )(rb, cbt)


def _shift_mats():
    t = np.zeros((NT, P, P), np.float32)
    for dy in range(KS):
        for dx in range(KS):
            for y in range(R):
                for x in range(R):
                    qy, qx = y + dy - 1, x + dx - 1
                    if 0 <= qy < R and 0 <= qx < R:
                        t[dy * KS + dx, qy * R + qx, y * R + x] = 1.0
    return jnp.asarray(t)


@jax.jit
def kernel(wrong_features, right_features, ref_boxes, curr_boxes, wq, bq, wk,
           bk, wv, bv, wo, bo, g1, b1, wgen_w, wgen_b, g2, b2, wf1, bf1, wf2,
           bf2, g3, b3):
    wf = wrong_features.reshape(K, D)
    rf = right_features.reshape(K, D)

    v = _mm_bias(rf, wv, bv.reshape(1, D))
    aor = _mm_bias_res(v, wo, bo.reshape(1, D), wf)
    x3, gap = _ln_gap(aor.reshape(K, C, P), g1.reshape(1, C, P),
                      b1.reshape(1, C, P))

    wgp = wgen_w.T.reshape(C, C, CG, NT).transpose(0, 3, 2, 1).reshape(C, -1)
    bgp = wgen_b.reshape(C, CG, NT).transpose(2, 1, 0).reshape(1, -1)
    dw = _gen(gap, wgp, bgp)

    dc3 = _conv_ln(x3, dw.reshape(K, NT, CG, C), _shift_mats(),
                   g2.reshape(1, C, P), b2.reshape(1, C, P))

    xf = dc3.reshape(K, D)
    h = _ffn1(xf, wf1, bf1.reshape(1, FFN))
    out = _ffn2_ln(h, wf2, bf2.reshape(1, D), xf, g3.reshape(1, D),
                   b3.reshape(1, D))

    cost = _cost(ref_boxes, curr_boxes.T)
    return out.reshape(K, C, R, R), cost
